# Initial kernel scaffold; baseline (speedup 1.0000x reference)
import jax
import jax.numpy as jnp
from jax.experimental import pallas as pl


def kernel(x, weight):
    raise NotImplementedError("write your pallas kernel here")


# speedup vs baseline: 1.4759x; 1.4759x over previous
import functools

import jax
import jax.numpy as jnp
from jax import lax
from jax.experimental import pallas as pl
from jax.experimental.pallas import tpu as pltpu
from jax.experimental.pallas import tpu_sc as plsc


@functools.lru_cache(maxsize=None)
def _make_gather(V, D, B):
  info = plsc.get_sparse_core_info()
  NC, NS = info.num_cores, info.num_subcores
  NW = NC * NS
  assert B % NW == 0
  b_per_w = B // NW
  C = 1600
  assert b_per_w % C == 0
  n_chunks = b_per_w // C

  mesh = plsc.VectorSubcoreMesh(core_axis_name="c", subcore_axis_name="s")

  @functools.partial(
      pl.kernel,
      mesh=mesh,
      out_type=jax.ShapeDtypeStruct((B, D), jnp.float32),
      scratch_types=[
          pltpu.VMEM((C,), jnp.int32),
          pltpu.VMEM((C, D), jnp.float32),
          pltpu.SemaphoreType.DMA,
      ],
      compiler_params=pltpu.CompilerParams(use_tc_tiling_on_sc=False),
  )
  def k(x_hbm, w_hbm, out_hbm, idx_v, rows_v, sem):
    wid = lax.axis_index("s") * NC + lax.axis_index("c")
    base0 = wid * b_per_w

    def body(i, carry):
      base = base0 + i * C
      pltpu.sync_copy(x_hbm.at[pl.ds(base, C)], idx_v)
      pltpu.async_copy(w_hbm.at[idx_v], rows_v, sem).wait()
      pltpu.sync_copy(rows_v, out_hbm.at[pl.ds(base, C)])
      return carry

    lax.fori_loop(0, n_chunks, body, 0)

  return k


def kernel(x, weight):
  Bt, T = x.shape
  V, D = weight.shape
  B = Bt * T
  xf = x.reshape(B).astype(jnp.int32)
  out = _make_gather(V, D, B)(xf, weight)
  return out.reshape(Bt, T, D)


# speedup vs baseline: 1.5003x; 1.0165x over previous
import functools

import jax
import jax.numpy as jnp
from jax import lax
from jax.experimental import pallas as pl
from jax.experimental.pallas import tpu as pltpu
from jax.experimental.pallas import tpu_sc as plsc

_NBUF = 3
_CHUNK = 1024


@functools.lru_cache(maxsize=None)
def _make_gather(V, D, B):
  info = plsc.get_sparse_core_info()
  NC, NS = info.num_cores, info.num_subcores
  NW = NC * NS
  assert B % NW == 0
  b_per_w = B // NW
  C = _CHUNK
  NBUF = _NBUF
  assert b_per_w % C == 0
  n_chunks = b_per_w // C
  assert b_per_w + NBUF * C * D <= 131000

  mesh = plsc.VectorSubcoreMesh(core_axis_name="c", subcore_axis_name="s")

  @functools.partial(
      pl.kernel,
      mesh=mesh,
      out_type=jax.ShapeDtypeStruct((B, D), jnp.float32),
      scratch_types=[
          pltpu.VMEM((b_per_w,), jnp.int32),
          pltpu.VMEM((NBUF, C, D), jnp.float32),
          pltpu.SemaphoreType.DMA((NBUF,)),
          pltpu.SemaphoreType.DMA((NBUF,)),
      ],
      compiler_params=pltpu.CompilerParams(use_tc_tiling_on_sc=False),
  )
  def k(x_hbm, w_hbm, out_hbm, idx_v, rows_v, gsem, ssem):
    wid = lax.axis_index("s") * NC + lax.axis_index("c")
    base0 = wid * b_per_w
    pltpu.sync_copy(x_hbm.at[pl.ds(base0, b_per_w)], idx_v)

    def gather(i, b):
      return pltpu.async_copy(
          w_hbm.at[idx_v.at[pl.ds(i * C, C)]], rows_v.at[b], gsem.at[b])

    def store(i, b):
      return pltpu.async_copy(
          rows_v.at[b], out_hbm.at[pl.ds(base0 + i * C, C)], ssem.at[b])

    g = [None] * NBUF
    s = [None] * NBUF
    for i in range(min(NBUF, n_chunks)):
      g[i] = gather(i, i)
    for i in range(n_chunks):
      b = i % NBUF
      g[b].wait()
      s[b] = store(i, b)
      nxt = i + NBUF
      if nxt < n_chunks:
        s[b].wait()
        g[b] = gather(nxt, b)
    for i in range(max(0, n_chunks - NBUF), n_chunks):
      s[i % NBUF].wait()

  return k


def kernel(x, weight):
  Bt, T = x.shape
  V, D = weight.shape
  B = Bt * T
  xf = x.reshape(B).astype(jnp.int32)
  out = _make_gather(V, D, B)(xf, weight)
  return out.reshape(Bt, T, D)
